# Initial kernel scaffold; baseline (speedup 1.0000x reference)
import jax
import jax.numpy as jnp
from jax.experimental import pallas as pl


def kernel(x, edge_index, batch, W_in, b_in, Wg, bg, gam, bet, Wgat, att_src, att_dst, b_gat, gam_gat, bet_gat, Wc1, bc1, Wc2, bc2, Wc3, bc3):
    raise NotImplementedError("write your pallas kernel here")


# speedup vs baseline: 1.0127x; 1.0127x over previous
import functools

import jax
import jax.numpy as jnp
from jax.experimental import pallas as pl
from jax.experimental.pallas import tpu as pltpu

N = 10000
E = 320000
F_IN = 128
HID = 128
HEADS = 4
G = 64
NCLS = 2
L = 3

ROWB = 1000


def _mm_relu_body(x_ref, w_ref, b_ref, o_ref):
    o_ref[...] = jnp.maximum(
        jnp.dot(x_ref[...], w_ref[...], preferred_element_type=jnp.float32)
        + b_ref[...], 0.0)


def _mm_relu(x, w, b):
    n, f = x.shape
    h = w.shape[1]
    return pl.pallas_call(
        _mm_relu_body,
        grid=(n // ROWB,),
        in_specs=[
            pl.BlockSpec((ROWB, f), lambda i: (i, 0)),
            pl.BlockSpec((f, h), lambda i: (0, 0)),
            pl.BlockSpec((1, h), lambda i: (0, 0)),
        ],
        out_specs=pl.BlockSpec((ROWB, h), lambda i: (i, 0)),
        out_shape=jax.ShapeDtypeStruct((n, h), jnp.float32),
    )(x, w, b.reshape(1, h))


def kernel(x, edge_index, batch, W_in, b_in, Wg, bg, gam, bet, Wgat,
           att_src, att_dst, b_gat, gam_gat, bet_gat, Wc1, bc1, Wc2, bc2,
           Wc3, bc3):
    src = edge_index[0]
    dst = edge_index[1]
    loop = jnp.arange(N, dtype=src.dtype)
    src2 = jnp.concatenate([src, loop])
    dst2 = jnp.concatenate([dst, loop])

    h = _mm_relu(x, W_in, b_in)

    deg = jnp.zeros((N,), jnp.float32).at[dst2].add(1.0)
    dinv = jax.lax.rsqrt(deg)
    enorm = dinv[src2] * dinv[dst2]

    def _bn(v, g, b):
        mu = v.mean(0)
        var = ((v - mu) ** 2).mean(0)
        return (v - mu) / jnp.sqrt(var + 1e-5) * g + b

    for i in range(L):
        idn = h
        m = h @ Wg[i]
        agg = jnp.zeros((N, HID), jnp.float32).at[dst2].add(
            m[src2] * enorm[:, None]) + bg[i]
        agg = _bn(agg, gam[i], bet[i])
        h = jax.nn.relu(agg) + idn

    idn = h
    hh = (h @ Wgat).reshape(N, HEADS, HID)
    a_s = (hh * att_src).sum(-1)
    a_d = (hh * att_dst).sum(-1)
    e = jax.nn.leaky_relu(a_s[src2] + a_d[dst2], 0.2)
    emax = jax.ops.segment_max(e, dst2, num_segments=N)
    emax = jnp.where(jnp.isfinite(emax), emax, 0.0)
    ee = jnp.exp(e - emax[dst2])
    den = jax.ops.segment_sum(ee, dst2, num_segments=N)
    alpha = ee / (den[dst2] + 1e-16)
    out = jax.ops.segment_sum(hh[src2] * alpha[:, :, None], dst2,
                              num_segments=N)
    out = out.mean(1) + b_gat
    out = _bn(out, gam_gat, bet_gat)
    h = jax.nn.elu(out) + idn

    cnt = jax.ops.segment_sum(jnp.ones((N,), jnp.float32), batch,
                              num_segments=G)
    x_mean = jax.ops.segment_sum(h, batch, num_segments=G) / jnp.maximum(
        cnt, 1.0)[:, None]
    x_max = jax.ops.segment_max(h, batch, num_segments=G)
    z = jnp.concatenate([x_mean, x_max], axis=1)
    z = jax.nn.relu(z @ Wc1 + bc1)
    z = jax.nn.relu(z @ Wc2 + bc2)
    return z @ Wc3 + bc3


# speedup vs baseline: 10.2142x; 10.0857x over previous
import functools

import jax
import jax.numpy as jnp
from jax import lax
from jax.experimental import pallas as pl
from jax.experimental.pallas import tpu as pltpu
from jax.experimental.pallas import tpu_sc as plsc

N = 10000
E = 320000
F_IN = 128
HID = 128
HEADS = 4
G = 64
NCLS = 2
L = 3

NC = 2
NS = 16
NW = NC * NS
LANES = 16

NPAD = 10112
ROWS_PT = NPAD // NS
EPT = 10240
EPAD = NW * EPT
CHW = 128
NCH = EPT // CHW

ROWB = 1000
NBLK = N // ROWB

@functools.cache
def _sc_mesh():
    return plsc.VectorSubcoreMesh(core_axis_name="c", subcore_axis_name="s",
                                  num_cores=NC, num_subcores=NS)
_SC_PARAMS = pltpu.CompilerParams(needs_layout_passes=False)


def _wid():
    return lax.axis_index("c") * NS + lax.axis_index("s")


def _sc_deg(dstp2):
    @functools.partial(
        pl.kernel,
        out_type=jax.ShapeDtypeStruct((NW * NPAD,), jnp.float32),
        mesh=_sc_mesh(),
        compiler_params=_SC_PARAMS,
        scratch_types=[
            pltpu.VMEM((NCH, CHW), jnp.int32),
            pltpu.VMEM((NPAD,), jnp.float32),
        ],
    )
    def k(dst_hbm, out_hbm, dst_loc, deg_loc):
        w = _wid()
        pltpu.sync_copy(dst_hbm.at[pl.ds(w * NCH, NCH)], dst_loc)
        zeros = jnp.zeros((LANES,), jnp.float32)

        def zero_body(i, _):
            deg_loc[pl.ds(i * LANES, LANES)] = zeros
            return _
        lax.fori_loop(0, NPAD // LANES, zero_body, None)

        ones = jnp.ones((LANES,), jnp.float32)

        def body(t, _):
            j = t // (CHW // LANES)
            kk = t % (CHW // LANES)
            idx = dst_loc[j, pl.ds(kk * LANES, LANES)]
            plsc.addupdate_scatter(deg_loc, [idx], ones)
            return _
        lax.fori_loop(0, EPT // LANES, body, None)
        pltpu.sync_copy(deg_loc, out_hbm.at[pl.ds(w * NPAD, NPAD)])

    return k(dstp2)


def _sc_gather_scatter(table, srcp2, dstp2, zrows):
    @functools.partial(
        pl.kernel,
        out_type=jax.ShapeDtypeStruct((NC, NPAD, HID), jnp.float32),
        mesh=_sc_mesh(),
        compiler_params=_SC_PARAMS,
        scratch_types=[
            pltpu.VMEM((NCH, CHW), jnp.int32),
            pltpu.VMEM((NCH, CHW), jnp.int32),
            pltpu.VMEM((CHW, HID), jnp.float32),
            pltpu.VMEM_SHARED((NPAD, HID), jnp.float32),
            pltpu.SemaphoreType.DMA,
        ],
    )
    def k(tab_hbm, src_hbm, dst_hbm, z_hbm, out_hbm,
          src_loc, dst_loc, rows, acc, sem):
        c = lax.axis_index("c")
        s = lax.axis_index("s")
        w = c * NS + s
        pltpu.sync_copy(src_hbm.at[pl.ds(w * NCH, NCH)], src_loc)
        pltpu.sync_copy(dst_hbm.at[pl.ds(w * NCH, NCH)], dst_loc)
        pltpu.sync_copy(z_hbm.at[pl.ds(s * ROWS_PT, ROWS_PT)],
                        acc.at[pl.ds(s * ROWS_PT, ROWS_PT)])
        plsc.subcore_barrier()

        def body(j, _):
            pltpu.async_copy(tab_hbm.at[src_loc.at[j]], rows, sem).wait()
            pltpu.sync_copy(rows, acc.at[dst_loc.at[j]], add=True)
            return _
        lax.fori_loop(0, NCH, body, None)
        plsc.subcore_barrier()
        pltpu.sync_copy(acc.at[pl.ds(s * ROWS_PT, ROWS_PT)],
                        out_hbm.at[c, pl.ds(s * ROWS_PT, ROWS_PT)])

    return k(table, srcp2, dstp2, zrows)


def _sc_gat_edge(a_sp, a_dp, srcp2, dstp2):
    @functools.partial(
        pl.kernel,
        out_type=jax.ShapeDtypeStruct((HEADS * EPAD,), jnp.float32),
        mesh=_sc_mesh(),
        compiler_params=_SC_PARAMS,
        scratch_types=[
            pltpu.VMEM((HEADS * NPAD,), jnp.float32),
            pltpu.VMEM((HEADS * NPAD,), jnp.float32),
            pltpu.VMEM((NCH, CHW), jnp.int32),
            pltpu.VMEM((NCH, CHW), jnp.int32),
            pltpu.VMEM((HEADS, CHW), jnp.float32),
        ],
    )
    def k(as_hbm, ad_hbm, src_hbm, dst_hbm, out_hbm,
          as_loc, ad_loc, src_loc, dst_loc, ev):
        w = _wid()
        pltpu.sync_copy(as_hbm, as_loc)
        pltpu.sync_copy(ad_hbm, ad_loc)
        pltpu.sync_copy(src_hbm.at[pl.ds(w * NCH, NCH)], src_loc)
        pltpu.sync_copy(dst_hbm.at[pl.ds(w * NCH, NCH)], dst_loc)

        def chunk(j, carry):
            def sub(kk, carry2):
                sl = pl.ds(kk * LANES, LANES)
                src16 = src_loc[j, sl]
                dst16 = dst_loc[j, sl]
                for h in range(HEADS):
                    sa = plsc.load_gather(as_loc, [src16 + h * NPAD])
                    da = plsc.load_gather(ad_loc, [dst16 + h * NPAD])
                    sm = sa + da
                    ev[h, sl] = jnp.maximum(sm, 0.2 * sm)
                return carry2
            lax.fori_loop(0, CHW // LANES, sub, None)
            for h in range(HEADS):
                pltpu.sync_copy(
                    ev.at[h],
                    out_hbm.at[pl.ds(h * EPAD + w * EPT + j * CHW, CHW)])
            return carry
        lax.fori_loop(0, NCH, chunk, None)

    return k(a_sp, a_dp, srcp2, dstp2)


def _sc_gat_emax(e_planes, dstp2):
    @functools.partial(
        pl.kernel,
        out_type=jax.ShapeDtypeStruct((NW * HEADS * NPAD,), jnp.float32),
        mesh=_sc_mesh(),
        compiler_params=_SC_PARAMS,
        scratch_types=[
            pltpu.VMEM((HEADS * NPAD,), jnp.float32),
            pltpu.VMEM((NCH, CHW), jnp.int32),
            pltpu.VMEM((HEADS, CHW), jnp.float32),
        ],
    )
    def k(e_hbm, dst_hbm, out_hbm, em_loc, dst_loc, ev):
        w = _wid()
        pltpu.sync_copy(dst_hbm.at[pl.ds(w * NCH, NCH)], dst_loc)
        neg = jnp.full((LANES,), -1e30, jnp.float32)

        def zb(i, carry):
            em_loc[pl.ds(i * LANES, LANES)] = neg
            return carry
        lax.fori_loop(0, HEADS * NPAD // LANES, zb, None)
        iota = lax.iota(jnp.int32, LANES)
        hsel = jnp.minimum(iota, HEADS - 1)

        def chunk(j, carry):
            for h in range(HEADS):
                pltpu.sync_copy(
                    e_hbm.at[pl.ds(h * EPAD + w * EPT + j * CHW, CHW)],
                    ev.at[h])

            def edge(t, carry2):
                dv = plsc.load_gather(dst_loc, [iota * 0 + j, iota * 0 + t])
                evv = plsc.load_gather(ev, [hsel, iota * 0 + t])
                idxm = hsel * NPAD + dv
                old = plsc.load_gather(em_loc, [idxm])
                plsc.store_scatter(em_loc, [idxm], jnp.maximum(old, evv))
                return carry2
            lax.fori_loop(0, CHW, edge, None)
            return carry
        lax.fori_loop(0, NCH, chunk, None)
        pltpu.sync_copy(em_loc,
                        out_hbm.at[pl.ds(w * HEADS * NPAD, HEADS * NPAD)])

    return k(e_planes, dstp2)


def _sc_gat_softmax(e_planes, emaxf, dstp2):
    @functools.partial(
        pl.kernel,
        out_type=[
            jax.ShapeDtypeStruct((HEADS * EPAD,), jnp.float32),
            jax.ShapeDtypeStruct((NW * HEADS * NPAD,), jnp.float32),
        ],
        mesh=_sc_mesh(),
        compiler_params=_SC_PARAMS,
        scratch_types=[
            pltpu.VMEM((HEADS * NPAD,), jnp.float32),
            pltpu.VMEM((HEADS * NPAD,), jnp.float32),
            pltpu.VMEM((NCH, CHW), jnp.int32),
            pltpu.VMEM((HEADS, CHW), jnp.float32),
            pltpu.VMEM((HEADS, CHW), jnp.float32),
        ],
    )
    def k(e_hbm, emax_hbm, dst_hbm, ee_hbm, den_hbm,
          em_loc, den_loc, dst_loc, ev, eev):
        w = _wid()
        pltpu.sync_copy(emax_hbm, em_loc)
        pltpu.sync_copy(dst_hbm.at[pl.ds(w * NCH, NCH)], dst_loc)
        zeros = jnp.zeros((LANES,), jnp.float32)

        def zb(i, carry):
            den_loc[pl.ds(i * LANES, LANES)] = zeros
            return carry
        lax.fori_loop(0, HEADS * NPAD // LANES, zb, None)

        def chunk(j, carry):
            for h in range(HEADS):
                pltpu.sync_copy(
                    e_hbm.at[pl.ds(h * EPAD + w * EPT + j * CHW, CHW)],
                    ev.at[h])

            def sub(kk, carry2):
                sl = pl.ds(kk * LANES, LANES)
                dst16 = dst_loc[j, sl]
                for h in range(HEADS):
                    em = plsc.load_gather(em_loc, [dst16 + h * NPAD])
                    ee = jnp.exp(ev[h, sl] - em)
                    eev[h, sl] = ee
                    plsc.addupdate_scatter(den_loc, [dst16 + h * NPAD], ee)
                return carry2
            lax.fori_loop(0, CHW // LANES, sub, None)
            for h in range(HEADS):
                pltpu.sync_copy(
                    eev.at[h],
                    ee_hbm.at[pl.ds(h * EPAD + w * EPT + j * CHW, CHW)])
            return carry
        lax.fori_loop(0, NCH, chunk, None)
        pltpu.sync_copy(den_loc,
                        den_hbm.at[pl.ds(w * HEADS * NPAD, HEADS * NPAD)])

    return k(e_planes, emaxf, dstp2)


def _sc_gat_aggregate(hh_h, ee_planes, srcp2, dstp2, zrows, head):
    @functools.partial(
        pl.kernel,
        out_type=jax.ShapeDtypeStruct((NC, NPAD, HID), jnp.float32),
        mesh=_sc_mesh(),
        compiler_params=_SC_PARAMS,
        scratch_types=[
            pltpu.VMEM((NCH, CHW), jnp.int32),
            pltpu.VMEM((NCH, CHW), jnp.int32),
            pltpu.VMEM((CHW, HID), jnp.float32),
            pltpu.VMEM((CHW,), jnp.float32),
            pltpu.VMEM_SHARED((NPAD, HID), jnp.float32),
            pltpu.SemaphoreType.DMA,
        ],
    )
    def k(tab_hbm, ee_hbm, src_hbm, dst_hbm, z_hbm, out_hbm,
          src_loc, dst_loc, rows, eev, acc, sem):
        c = lax.axis_index("c")
        s = lax.axis_index("s")
        w = c * NS + s
        pltpu.sync_copy(src_hbm.at[pl.ds(w * NCH, NCH)], src_loc)
        pltpu.sync_copy(dst_hbm.at[pl.ds(w * NCH, NCH)], dst_loc)
        pltpu.sync_copy(z_hbm.at[pl.ds(s * ROWS_PT, ROWS_PT)],
                        acc.at[pl.ds(s * ROWS_PT, ROWS_PT)])
        plsc.subcore_barrier()
        iota = lax.iota(jnp.int32, LANES)

        def body(j, carry):
            pltpu.async_copy(tab_hbm.at[src_loc.at[j]], rows, sem).wait()
            pltpu.sync_copy(
                ee_hbm.at[pl.ds(head * EPAD + w * EPT + j * CHW, CHW)], eev)

            def edge(t, carry2):
                wsp = plsc.load_gather(eev, [iota * 0 + t])
                for kk in range(HID // LANES):
                    sl = pl.ds(kk * LANES, LANES)
                    rows[t, sl] = rows[t, sl] * wsp
                return carry2
            lax.fori_loop(0, CHW, edge, None)
            pltpu.sync_copy(rows, acc.at[dst_loc.at[j]], add=True)
            return carry
        lax.fori_loop(0, NCH, body, None)
        plsc.subcore_barrier()
        pltpu.sync_copy(acc.at[pl.ds(s * ROWS_PT, ROWS_PT)],
                        out_hbm.at[c, pl.ds(s * ROWS_PT, ROWS_PT)])

    return k(hh_h, ee_planes, srcp2, dstp2, zrows)


def _tc_mm_relu(x, w, b):
    def body(x_ref, w_ref, b_ref, o_ref):
        o_ref[...] = jnp.maximum(
            jnp.dot(x_ref[...], w_ref[...],
                    preferred_element_type=jnp.float32) + b_ref[...], 0.0)
    n, f = x.shape
    h = w.shape[1]
    return pl.pallas_call(
        body,
        grid=(n // ROWB,),
        in_specs=[
            pl.BlockSpec((ROWB, f), lambda i: (i, 0)),
            pl.BlockSpec((f, h), lambda i: (0, 0)),
            pl.BlockSpec((1, h), lambda i: (0, 0)),
        ],
        out_specs=pl.BlockSpec((ROWB, h), lambda i: (i, 0)),
        out_shape=jax.ShapeDtypeStruct((n, h), jnp.float32),
    )(x, w, b.reshape(1, h))


def _tc_mm_scale(x, w, scale):
    def body(x_ref, w_ref, s_ref, o_ref):
        o_ref[...] = jnp.dot(x_ref[...], w_ref[...],
                             preferred_element_type=jnp.float32) * s_ref[...]
    return pl.pallas_call(
        body,
        grid=(NBLK,),
        in_specs=[
            pl.BlockSpec((ROWB, HID), lambda i: (i, 0)),
            pl.BlockSpec((HID, HID), lambda i: (0, 0)),
            pl.BlockSpec((ROWB, HID), lambda i: (i, 0)),
        ],
        out_specs=pl.BlockSpec((ROWB, HID), lambda i: (i, 0)),
        out_shape=jax.ShapeDtypeStruct((N, HID), jnp.float32),
    )(x, w, scale)


def _tc_gcn_agg(p0, p1, mt, dinvb, bg_i):
    def body(p0_ref, p1_ref, mt_ref, d_ref, b_ref, agg_ref, st_ref):
        i = pl.program_id(0)
        agg = ((p0_ref[...] + p1_ref[...] + mt_ref[...]) * d_ref[...]
               + b_ref[...])
        agg_ref[...] = agg

        @pl.when(i == 0)
        def _():
            st_ref[...] = jnp.zeros_like(st_ref)
        st_ref[0:1, :] = st_ref[0:1, :] + jnp.sum(agg, 0, keepdims=True)
        st_ref[1:2, :] = st_ref[1:2, :] + jnp.sum(agg * agg, 0,
                                                  keepdims=True)
    return pl.pallas_call(
        body,
        grid=(NBLK,),
        in_specs=[pl.BlockSpec((ROWB, HID), lambda i: (i, 0))] * 4
        + [pl.BlockSpec((1, HID), lambda i: (0, 0))],
        out_specs=[
            pl.BlockSpec((ROWB, HID), lambda i: (i, 0)),
            pl.BlockSpec((8, HID), lambda i: (0, 0)),
        ],
        out_shape=[
            jax.ShapeDtypeStruct((N, HID), jnp.float32),
            jax.ShapeDtypeStruct((8, HID), jnp.float32),
        ],
    )(p0, p1, mt, dinvb, bg_i.reshape(1, HID))


def _tc_bn_act(agg, stats, g, b, hprev, act):
    def body(a_ref, st_ref, g_ref, b_ref, hp_ref, o_ref):
        mu = st_ref[0:1, :] / N
        var = st_ref[1:2, :] / N - mu * mu
        rstd = lax.rsqrt(var + 1e-5)
        y = (a_ref[...] - mu) * rstd * g_ref[...] + b_ref[...]
        if act == 'relu':
            y = jnp.maximum(y, 0.0)
        else:
            y = jnp.where(y > 0.0, y, jnp.exp(jnp.minimum(y, 0.0)) - 1.0)
        o_ref[...] = y + hp_ref[...]
    return pl.pallas_call(
        body,
        grid=(NBLK,),
        in_specs=[
            pl.BlockSpec((ROWB, HID), lambda i: (i, 0)),
            pl.BlockSpec((8, HID), lambda i: (0, 0)),
            pl.BlockSpec((1, HID), lambda i: (0, 0)),
            pl.BlockSpec((1, HID), lambda i: (0, 0)),
            pl.BlockSpec((ROWB, HID), lambda i: (i, 0)),
        ],
        out_specs=pl.BlockSpec((ROWB, HID), lambda i: (i, 0)),
        out_shape=jax.ShapeDtypeStruct((N, HID), jnp.float32),
    )(agg, stats, g.reshape(1, HID), b.reshape(1, HID), hprev)


def _tc_gat_head(h, Wgat, att_src_p, att_dst_p):
    def body(x_ref, w_ref, as_ref, ad_ref, hh0, hh1, hh2, hh3, aux_ref):
        hh_refs = (hh0, hh1, hh2, hh3)
        sa_cols = []
        da_cols = []
        for hd in range(HEADS):
            y = jnp.dot(x_ref[...], w_ref[:, hd * HID:(hd + 1) * HID],
                        preferred_element_type=jnp.float32)
            hh_refs[hd][...] = y
            sa_cols.append(jnp.sum(y * as_ref[hd:hd + 1, :], axis=1,
                                   keepdims=True))
            da_cols.append(jnp.sum(y * ad_ref[hd:hd + 1, :], axis=1,
                                   keepdims=True))
        a_s = jnp.concatenate(sa_cols, axis=1)
        a_d = jnp.concatenate(da_cols, axis=1)
        es = a_s + a_d
        es = jnp.maximum(es, 0.2 * es)
        zpad = jnp.zeros((a_s.shape[0], HID - 3 * HEADS), jnp.float32)
        aux_ref[...] = jnp.concatenate([a_s, a_d, es, zpad], axis=1)

    return pl.pallas_call(
        body,
        grid=(NBLK,),
        in_specs=[
            pl.BlockSpec((ROWB, HID), lambda i: (i, 0)),
            pl.BlockSpec((HID, HEADS * HID), lambda i: (0, 0)),
            pl.BlockSpec((8, HID), lambda i: (0, 0)),
            pl.BlockSpec((8, HID), lambda i: (0, 0)),
        ],
        out_specs=[pl.BlockSpec((ROWB, HID), lambda i: (i, 0))] * 5,
        out_shape=[jax.ShapeDtypeStruct((N, HID), jnp.float32)] * 5,
    )(h, Wgat, att_src_p, att_dst_p)


def _tc_gat_final(parts, hhs, aux2, b_gat):
    def body(p00, p01, p10, p11, p20, p21, p30, p31,
             hh0, hh1, hh2, hh3, aux_ref, b_ref, o_ref, st_ref):
        i = pl.program_id(0)
        ps = ((p00, p01), (p10, p11), (p20, p21), (p30, p31))
        hh = (hh0, hh1, hh2, hh3)
        acc = None
        for hd in range(HEADS):
            rden = aux_ref[:, hd:hd + 1]
            ees = aux_ref[:, HEADS + hd:HEADS + hd + 1]
            term = (ps[hd][0][...] + ps[hd][1][...]
                    + ees * hh[hd][...]) * rden
            acc = term if acc is None else acc + term
        out = acc * (1.0 / HEADS) + b_ref[...]
        o_ref[...] = out

        @pl.when(i == 0)
        def _():
            st_ref[...] = jnp.zeros_like(st_ref)
        st_ref[0:1, :] = st_ref[0:1, :] + jnp.sum(out, 0, keepdims=True)
        st_ref[1:2, :] = st_ref[1:2, :] + jnp.sum(out * out, 0,
                                                  keepdims=True)

    return pl.pallas_call(
        body,
        grid=(NBLK,),
        in_specs=[pl.BlockSpec((ROWB, HID), lambda i: (i, 0))] * 13
        + [pl.BlockSpec((1, HID), lambda i: (0, 0))],
        out_specs=[
            pl.BlockSpec((ROWB, HID), lambda i: (i, 0)),
            pl.BlockSpec((8, HID), lambda i: (0, 0)),
        ],
        out_shape=[
            jax.ShapeDtypeStruct((N, HID), jnp.float32),
            jax.ShapeDtypeStruct((8, HID), jnp.float32),
        ],
    )(*parts, *hhs, aux2, b_gat.reshape(1, HID))


def _tc_pool(h, bcolb):
    def body(h_ref, bc_ref, sum_ref, cnt_ref, max_ref):
        i = pl.program_id(0)

        @pl.when(i == 0)
        def _():
            sum_ref[...] = jnp.zeros_like(sum_ref)
            cnt_ref[...] = jnp.zeros_like(cnt_ref)
            max_ref[...] = jnp.full_like(max_ref, -jnp.inf)

        bcol = bc_ref[:, 0:1]
        hv = h_ref[...]
        srows, crows, mrows = [], [], []
        for g in range(G):
            mrow = (bcol == g)
            mf = mrow.astype(jnp.float32)
            srows.append(jnp.sum(jnp.where(mrow, hv, 0.0), axis=0,
                                 keepdims=True))
            crows.append(jnp.broadcast_to(
                jnp.sum(mf, axis=0, keepdims=True), (1, HID)))
            mrows.append(jnp.max(jnp.where(mrow, hv, -jnp.inf), axis=0,
                                 keepdims=True))
        sum_ref[...] = sum_ref[...] + jnp.concatenate(srows, axis=0)
        cnt_ref[...] = cnt_ref[...] + jnp.concatenate(crows, axis=0)
        max_ref[...] = jnp.maximum(max_ref[...],
                                   jnp.concatenate(mrows, axis=0))

    return pl.pallas_call(
        body,
        grid=(NBLK,),
        in_specs=[
            pl.BlockSpec((ROWB, HID), lambda i: (i, 0)),
            pl.BlockSpec((ROWB, HID), lambda i: (i, 0)),
        ],
        out_specs=[pl.BlockSpec((G, HID), lambda i: (0, 0))] * 3,
        out_shape=[jax.ShapeDtypeStruct((G, HID), jnp.float32)] * 3,
    )(h, bcolb)


def _tc_mlp(sums, cnts, maxs, W1a, W1b, b1, W2p, b2p, W3p, b3p):
    def body(s_ref, c_ref, m_ref, w1a, w1b, b1r, w2, b2r, w3, b3r, o_ref):
        mean = s_ref[...] / jnp.maximum(c_ref[...], 1.0)
        z1 = jnp.maximum(
            jnp.dot(mean, w1a[...], preferred_element_type=jnp.float32)
            + jnp.dot(m_ref[...], w1b[...],
                      preferred_element_type=jnp.float32) + b1r[...], 0.0)
        z2 = jnp.maximum(
            jnp.dot(z1, w2[...], preferred_element_type=jnp.float32)
            + b2r[...], 0.0)
        o_ref[...] = jnp.dot(z2, w3[...],
                             preferred_element_type=jnp.float32) + b3r[...]

    specs = [pl.BlockSpec((G, HID), lambda: (0, 0))] * 3 + [
        pl.BlockSpec((HID, HID), lambda: (0, 0)),
        pl.BlockSpec((HID, HID), lambda: (0, 0)),
        pl.BlockSpec((1, HID), lambda: (0, 0)),
        pl.BlockSpec((HID, HID), lambda: (0, 0)),
        pl.BlockSpec((1, HID), lambda: (0, 0)),
        pl.BlockSpec((HID, HID), lambda: (0, 0)),
        pl.BlockSpec((1, HID), lambda: (0, 0)),
    ]
    return pl.pallas_call(
        body,
        in_specs=specs,
        out_specs=pl.BlockSpec((G, HID), lambda: (0, 0)),
        out_shape=jax.ShapeDtypeStruct((G, HID), jnp.float32),
    )(sums, cnts, maxs, W1a, W1b, b1.reshape(1, HID), W2p,
      b2p.reshape(1, HID), W3p, b3p.reshape(1, HID))


def kernel(x, edge_index, batch, W_in, b_in, Wg, bg, gam, bet, Wgat,
           att_src, att_dst, b_gat, gam_gat, bet_gat, Wc1, bc1, Wc2, bc2,
           Wc3, bc3):
    src = edge_index[0]
    dst = edge_index[1]
    pad = EPAD - E
    srcp2 = jnp.concatenate([src, jnp.zeros((pad,), jnp.int32)]
                            ).reshape(EPAD // CHW, CHW)
    dstp2 = jnp.concatenate([dst, jnp.full((pad,), N, jnp.int32)]
                            ).reshape(EPAD // CHW, CHW)
    zrows = jnp.zeros((NPAD, HID), jnp.float32)

    h = _tc_mm_relu(x, W_in, b_in)

    degp = _sc_deg(dstp2).reshape(NW, NPAD)
    deg = degp.sum(0)[:N] + 1.0
    dinv = lax.rsqrt(deg)
    dinvb = jnp.broadcast_to(dinv[:, None], (N, HID))

    for i in range(L):
        idn = h
        mt = _tc_mm_scale(h, Wg[i], dinvb)
        part = _sc_gather_scatter(mt, srcp2, dstp2, zrows)
        agg, stats = _tc_gcn_agg(part[0, :N], part[1, :N], mt, dinvb, bg[i])
        h = _tc_bn_act(agg, stats, gam[i], bet[i], idn, 'relu')

    idn = h
    att_src_p = jnp.pad(att_src, ((0, 8 - HEADS), (0, 0)))
    att_dst_p = jnp.pad(att_dst, ((0, 8 - HEADS), (0, 0)))
    hh0, hh1, hh2, hh3, aux1 = _tc_gat_head(h, Wgat, att_src_p, att_dst_p)
    hhs = (hh0, hh1, hh2, hh3)

    def _plane(v):
        return jnp.pad(v.T, ((0, 0), (0, NPAD - N))).reshape(-1)

    a_sp = _plane(aux1[:, 0:HEADS])
    a_dp = _plane(aux1[:, HEADS:2 * HEADS])
    es_p = _plane(aux1[:, 2 * HEADS:3 * HEADS])

    e_planes = _sc_gat_edge(a_sp, a_dp, srcp2, dstp2)
    emaxp = _sc_gat_emax(e_planes, dstp2)
    emaxf = jnp.maximum(emaxp.reshape(NW, HEADS * NPAD).max(0), es_p)
    ee_planes, denp = _sc_gat_softmax(e_planes, emaxf, dstp2)
    ee_self = jnp.exp(es_p - emaxf)
    den = denp.reshape(NW, HEADS * NPAD).sum(0) + ee_self
    rden = 1.0 / (den + 1e-16)

    def _unplane(v):
        return v.reshape(HEADS, NPAD)[:, :N].T

    aux2 = jnp.concatenate(
        [_unplane(rden), _unplane(ee_self),
         jnp.zeros((N, HID - 2 * HEADS), jnp.float32)], axis=1)

    parts = []
    for hd in range(HEADS):
        p = _sc_gat_aggregate(hhs[hd], ee_planes, srcp2, dstp2, zrows, hd)
        parts.extend([p[0, :N], p[1, :N]])
    gat_raw, gstats = _tc_gat_final(parts, hhs, aux2, b_gat)
    h = _tc_bn_act(gat_raw, gstats, gam_gat, bet_gat, idn, 'elu')

    bf = batch.astype(jnp.float32)
    bcolb = jnp.broadcast_to(bf[:, None], (N, HID))
    sums, cnts, maxs = _tc_pool(h, bcolb)

    W1a = Wc1[:HID]
    W1b = Wc1[HID:]
    W2p = jnp.pad(Wc2, ((0, 0), (0, HID - HID // 2)))
    b2p = jnp.pad(bc2, ((0, HID - HID // 2),))
    W3p = jnp.pad(Wc3, ((0, HID - HID // 2), (0, HID - NCLS)))
    b3p = jnp.pad(bc3, ((0, HID - NCLS),))
    z = _tc_mlp(sums, cnts, maxs, W1a, W1b, bc1, W2p, b2p, W3p, b3p)
    return z[:, :NCLS]
